# baseline (speedup 1.0000x reference)
import functools

import jax
import jax.numpy as jnp
from jax import lax
from jax.experimental import pallas as pl
from jax.experimental.pallas import tpu as pltpu
from jax.experimental.pallas import tpu_sc as plsc

B, L = 4096, 50
D = 64
NC, NS = 2, 16
NW = NC * NS
BPW = B // NW
K = 4
CC = K * L
NCHUNK = BPW // K

_mesh = plsc.VectorSubcoreMesh(core_axis_name="c", subcore_axis_name="s")


@functools.partial(
    pl.kernel,
    mesh=_mesh,
    out_type=jax.ShapeDtypeStruct((B, L, D), jnp.float32),
    compiler_params=pltpu.CompilerParams(use_tc_tiling_on_sc=False),
    scratch_types=[
        pltpu.VMEM((BPW, L), jnp.int32),
        pltpu.VMEM((BPW, L), jnp.int32),
        pltpu.VMEM((BPW, L), jnp.int32),
        pltpu.VMEM((BPW * L,), jnp.int32),
        pltpu.VMEM((BPW * L,), jnp.int32),
        pltpu.VMEM((BPW * L,), jnp.int32),
        pltpu.VMEM((2, CC, D), jnp.float32),
        pltpu.VMEM((2, CC, D), jnp.float32),
        pltpu.VMEM((2, CC, D), jnp.float32),
        pltpu.SemaphoreType.DMA,
        pltpu.SemaphoreType.DMA,
        pltpu.SemaphoreType.DMA,
        pltpu.SemaphoreType.DMA,
        pltpu.SemaphoreType.DMA,
        pltpu.SemaphoreType.DMA,
    ],
)
def _triple_embed(oid, tid, cid, t1, t2, t3, out,
                  i1, i2, i3, c1, c2, c3, b1, b2, b3,
                  sa1, sa2, sa3, sb1, sb2, sb3):
    wid = lax.axis_index("s") * NC + lax.axis_index("c")
    wb = wid * BPW

    pltpu.sync_copy(oid.at[pl.ds(wb, BPW)], i1)
    pltpu.sync_copy(tid.at[pl.ds(wb, BPW)], i2)
    pltpu.sync_copy(cid.at[pl.ds(wb, BPW)], i3)

    def compact(r, cc):
        base = r * L
        for src, dst in ((i1, c1), (i2, c2), (i3, c3)):
            for off in (0, 16, 32, 34):
                dst[pl.ds(base + off, 16)] = src[r, pl.ds(off, 16)]
        return cc

    lax.fori_loop(0, BPW, compact, 0)

    sems = ((sa1, sa2, sa3), (sb1, sb2, sb3))

    def fire(c, k):
        s1, s2, s3 = sems[k]
        isl = pl.ds(c * CC, CC)
        pltpu.async_copy(t1.at[c1.at[isl]], b1.at[k], s1)
        pltpu.async_copy(t2.at[c2.at[isl]], b2.at[k], s2)
        pltpu.async_copy(t3.at[c3.at[isl]], b3.at[k], s3)

    def drain(c, k):
        s1, s2, s3 = sems[k]
        isl = pl.ds(c * CC, CC)
        pltpu.make_async_copy(t1.at[c1.at[isl]], b1.at[k], s1).wait()
        pltpu.make_async_copy(t2.at[c2.at[isl]], b2.at[k], s2).wait()
        pltpu.make_async_copy(t3.at[c3.at[isl]], b3.at[k], s3).wait()

        def row(r, cc):
            for j in range(D // 16):
                sl = pl.ds(j * 16, 16)
                b1[k, r, sl] = b1[k, r, sl] + b2[k, r, sl] + b3[k, r, sl]
            return cc

        lax.fori_loop(0, CC, row, 0)
        for j in range(K):
            pltpu.sync_copy(b1.at[k, pl.ds(j * L, L)], out.at[wb + c * K + j])

    fire(0, 0)

    def body(h, carry):
        ca = 2 * h
        fire(ca + 1, 1)
        drain(ca, 0)

        @pl.when(ca + 2 < NCHUNK)
        def _():
            fire(ca + 2, 0)

        drain(ca + 1, 1)
        return carry

    lax.fori_loop(0, NCHUNK // 2, body, 0)


def kernel(out_ids, tree_ids, ctx_ids, out_table, tree_table, ctx_table):
    return _triple_embed(out_ids.astype(jnp.int32), tree_ids.astype(jnp.int32),
                         ctx_ids.astype(jnp.int32), out_table, tree_table, ctx_table)
